# baseline (speedup 1.0000x reference)
import functools
import jax
import jax.numpy as jnp
from jax import lax
from jax.experimental import pallas as pl
from jax.experimental.pallas import tpu as pltpu
from jax.experimental.pallas import tpu_sc as plsc

D_MODEL = 1024
SCALE = 32.0
NC, NS, L = 2, 16, 16
NW = NC * NS
CHUNK = 32


def _body(ids_hbm, table_hbm, out_hbm, idx_v, buf0, buf1, gsem0, gsem1, ssem0, ssem1):
    bufs = (buf0, buf1)
    gsems = (gsem0, gsem1)
    ssems = (ssem0, ssem1)
    B = ids_hbm.shape[0]
    bpw = B // NW
    n = bpw // CHUNK
    wid = lax.axis_index("s") * NC + lax.axis_index("c")
    base = wid * bpw

    pltpu.sync_copy(ids_hbm.at[pl.ds(base, bpw)], idx_v)

    def gather(c, b):
        return pltpu.make_async_copy(
            table_hbm.at[idx_v.at[pl.ds(c * CHUNK, CHUNK)]], bufs[b], gsems[b]
        )

    def store(c, b):
        return pltpu.make_async_copy(
            bufs[b], out_hbm.at[pl.ds(base + c * CHUNK, CHUNK)], ssems[b]
        )

    gather(0, 0).start()

    def scale_buf(buf):
        @plsc.parallel_loop(0, CHUNK, 1)
        def _(j):
            for k in range(D_MODEL // L):
                sl = pl.ds(k * L, L)
                buf[j, sl] = buf[j, sl] * SCALE

    def outer(t, _):
        for b in range(2):
            c = 2 * t + b

            gather(c, b).wait()
            scale_buf(bufs[b])

            @pl.when(c >= 1)
            def _():
                store(c - 1, 1 - b).wait()

            @pl.when(c + 1 < n)
            def _():
                gather(c + 1, 1 - b).start()

            store(c, b).start()
        return _

    lax.fori_loop(0, n // 2, outer, None)
    store(n - 1, (n - 1) % 2).wait()


def kernel(input_ids, table):
    B = input_ids.size
    ids_flat = input_ids.reshape(B)
    mesh = plsc.VectorSubcoreMesh(
        core_axis_name="c", subcore_axis_name="s", num_cores=NC, num_subcores=NS
    )
    out = pl.kernel(
        _body,
        out_type=jax.ShapeDtypeStruct((B, D_MODEL), jnp.float32),
        mesh=mesh,
        scratch_types=[
            pltpu.VMEM((B // NW,), jnp.int32),
            pltpu.VMEM((CHUNK, D_MODEL), jnp.float32),
            pltpu.VMEM((CHUNK, D_MODEL), jnp.float32),
            pltpu.SemaphoreType.DMA,
            pltpu.SemaphoreType.DMA,
            pltpu.SemaphoreType.DMA,
            pltpu.SemaphoreType.DMA,
        ],
    )(ids_flat, table)
    return out.reshape(input_ids.shape[0], input_ids.shape[1], D_MODEL)


# speedup vs baseline: 1.3266x; 1.3266x over previous
import functools
import jax
import jax.numpy as jnp
from jax import lax
from jax.experimental import pallas as pl
from jax.experimental.pallas import tpu as pltpu
from jax.experimental.pallas import tpu_sc as plsc

D_MODEL = 1024
SCALE = 32.0
NC, NS, L = 2, 16, 16
NW = NC * NS
CHUNK = 16
NBUF = 4


def _body(ids_hbm, table_hbm, out_hbm, idx_v, buf0, buf1, buf2, buf3,
          gsem0, gsem1, gsem2, gsem3, ssem0, ssem1, ssem2, ssem3):
    bufs = (buf0, buf1, buf2, buf3)
    gsems = (gsem0, gsem1, gsem2, gsem3)
    ssems = (ssem0, ssem1, ssem2, ssem3)
    B = ids_hbm.shape[0]
    bpw = B // NW
    n = bpw // CHUNK
    wid = lax.axis_index("s") * NC + lax.axis_index("c")
    base = wid * bpw

    pltpu.sync_copy(ids_hbm.at[pl.ds(base, bpw)], idx_v)

    def gather(c, b):
        return pltpu.make_async_copy(
            table_hbm.at[idx_v.at[pl.ds(c * CHUNK, CHUNK)]], bufs[b], gsems[b]
        )

    def store(c, b):
        return pltpu.make_async_copy(
            bufs[b], out_hbm.at[pl.ds(base + c * CHUNK, CHUNK)], ssems[b]
        )

    gather(0, 0).start()
    gather(1, 1).start()

    def scale_buf(buf):
        @plsc.parallel_loop(0, CHUNK, 1)
        def _(j):
            for k in range(D_MODEL // L):
                sl = pl.ds(k * L, L)
                buf[j, sl] = buf[j, sl] * SCALE

    def outer(t, _):
        for b in range(NBUF):
            c = NBUF * t + b
            nb = (b + 2) % NBUF

            @pl.when(c >= 2)
            def _():
                store(c - 2, nb).wait()

            @pl.when(c + 2 < n)
            def _():
                gather(c + 2, nb).start()

            gather(c, b).wait()
            scale_buf(bufs[b])
            store(c, b).start()
        return _

    lax.fori_loop(0, n // NBUF, outer, None)
    store(n - 2, (n - 2) % NBUF).wait()
    store(n - 1, (n - 1) % NBUF).wait()


def kernel(input_ids, table):
    B = input_ids.size
    ids_flat = input_ids.reshape(B)
    mesh = plsc.VectorSubcoreMesh(
        core_axis_name="c", subcore_axis_name="s", num_cores=NC, num_subcores=NS
    )
    out = pl.kernel(
        _body,
        out_type=jax.ShapeDtypeStruct((B, D_MODEL), jnp.float32),
        mesh=mesh,
        scratch_types=(
            [pltpu.VMEM((B // NW,), jnp.int32)]
            + [pltpu.VMEM((CHUNK, D_MODEL), jnp.float32)] * NBUF
            + [pltpu.SemaphoreType.DMA] * (2 * NBUF)
        ),
    )(ids_flat, table)
    return out.reshape(input_ids.shape[0], input_ids.shape[1], D_MODEL)
